# baseline (speedup 1.0000x reference)
import functools

import jax
import jax.numpy as jnp
from jax import lax
from jax.experimental import pallas as pl
from jax.experimental.pallas import tpu as pltpu
from jax.experimental.pallas import tpu_sc as plsc

_D = 13
_NW = 32
_CH = 3200


def _sc_bias_kernel(
    CH, n_chunks, at_hbm, wvc_hbm, out_hbm,
    b0a, b1a, b0b, b1b, acc_buf, wv_buf, sem_a, sem_b, sem_a2, sem_b2,
):
    wid = lax.axis_index("s") * 2 + lax.axis_index("c")
    n_t = (n_chunks - wid + _NW - 1) // _NW

    pltpu.sync_copy(wvc_hbm, wv_buf)
    wv_all = wv_buf[...]

    def _splat(d):
        idx = jnp.full((16,), d, jnp.int32)
        return wv_all.at[idx].get(mode="promise_in_bounds")

    wvecs = [_splat(d) for d in range(_D)]
    cvec = _splat(_D)

    pairs = ((b0a, b1a, sem_a, sem_a2), (b0b, b1b, sem_b, sem_b2))

    def in_copies(t, pair):
        b0, b1, sem, sem2 = pair
        e0 = (t * _NW + wid) * CH
        return (
            pltpu.make_async_copy(at_hbm.at[pl.ds(0, 8), pl.ds(e0, CH)], b0, sem),
            pltpu.make_async_copy(at_hbm.at[pl.ds(8, _D - 8), pl.ds(e0, CH)], b1, sem2),
        )

    @pl.when(n_t > 0)
    def _():
        for cp in in_copies(0, pairs[0]):
            cp.start()

    per_w = -(-n_chunks // _NW)
    n_outer = -(-per_w // 2)

    def outer(k, _):
        for b in range(2):
            t = k * 2 + b

            @pl.when(t + 1 < n_t)
            def _():
                for cp in in_copies(t + 1, pairs[1 - b]):
                    cp.start()

            @pl.when(t < n_t)
            def _():
                b0, b1 = pairs[b][0], pairs[b][1]
                for cp in in_copies(t, pairs[b]):
                    cp.wait()

                def group(g):
                    acc = cvec
                    for d in range(8):
                        acc = acc + b0[d, pl.ds(g, 16)] * wvecs[d]
                    for d in range(_D - 8):
                        acc = acc + b1[d, pl.ds(g, 16)] * wvecs[8 + d]
                    acc_buf[pl.ds(g, 16)] = acc

                plsc.parallel_loop(0, CH, step=16, unroll=4)(group)
                e0 = (t * _NW + wid) * CH
                pltpu.sync_copy(acc_buf, out_hbm.at[pl.ds(e0, CH)])

        return 0

    lax.fori_loop(0, n_outer, outer, 0)


def _sc_part(At, wvc, E_sc):
    n_chunks = E_sc // _CH
    mesh = plsc.VectorSubcoreMesh(
        core_axis_name="c", subcore_axis_name="s", num_cores=2, num_subcores=16
    )
    k = pl.kernel(
        functools.partial(_sc_bias_kernel, _CH, n_chunks),
        mesh=mesh,
        out_type=jax.ShapeDtypeStruct((E_sc,), jnp.float32),
        scratch_types=[
            pltpu.VMEM((8, _CH), jnp.float32),
            pltpu.VMEM((_D - 8, _CH), jnp.float32),
            pltpu.VMEM((8, _CH), jnp.float32),
            pltpu.VMEM((_D - 8, _CH), jnp.float32),
            pltpu.VMEM((_CH,), jnp.float32),
            pltpu.VMEM((16,), jnp.float32),
            pltpu.SemaphoreType.DMA,
            pltpu.SemaphoreType.DMA,
            pltpu.SemaphoreType.DMA,
            pltpu.SemaphoreType.DMA,
        ],
        compiler_params=pltpu.CompilerParams(needs_layout_passes=False),
    )
    return k(At, wvc)


def _tc_body(a_ref, w_ref, c_ref, o_ref):
    o_ref[...] = jnp.sum(a_ref[...] * w_ref[...], axis=0) + c_ref[0, 0]


def _tc_part(At, wcol, c2d, E, blk0, n_blk, BLKL):
    D = At.shape[0]
    return pl.pallas_call(
        _tc_body,
        grid=(n_blk,),
        in_specs=[
            pl.BlockSpec((D, BLKL), lambda i: (0, i + blk0)),
            pl.BlockSpec((D, 1), lambda i: (0, 0)),
            pl.BlockSpec(memory_space=pltpu.SMEM),
        ],
        out_specs=pl.BlockSpec((BLKL,), lambda i: (i,)),
        out_shape=jax.ShapeDtypeStruct((n_blk * BLKL,), jnp.float32),
        compiler_params=pltpu.CompilerParams(
            dimension_semantics=("arbitrary",),
        ),
    )(At, wcol, c2d)


def kernel(edge_attr, W_edge, b_edge, edge_index, n_nodes, batch):
    E, D = edge_attr.shape
    if E == 0:
        return jnp.zeros((0,), dtype=jnp.float32)

    wv = jnp.mean(W_edge, axis=1)
    c = jnp.mean(b_edge)
    wvc = jnp.concatenate([wv, c[None], jnp.zeros((16 - D - 1,), jnp.float32)])
    wcol = wv[:, None]
    c2d = c.reshape(1, 1)

    At = edge_attr.T

    BLKL = 128
    for cand in range(131072, 127, -128):
        if E % cand == 0 and cand % _CH == 0:
            BLKL = cand
            break
    n_blk = E // BLKL
    k_sc = n_blk // 4

    if BLKL % _CH or k_sc == 0 or k_sc == n_blk:
        return _tc_part(At, wcol, c2d, E, 0, n_blk, BLKL)

    out_sc = _sc_part(At, wvc, k_sc * BLKL)
    out_tc = _tc_part(At, wcol, c2d, E, k_sc, n_blk - k_sc, BLKL)
    return jnp.concatenate([out_sc, out_tc])


# speedup vs baseline: 1.0168x; 1.0168x over previous
import functools

import jax
import jax.numpy as jnp
from jax import lax
from jax.experimental import pallas as pl
from jax.experimental.pallas import tpu as pltpu
from jax.experimental.pallas import tpu_sc as plsc

_D = 13
_NW = 32
_CH = 3200


def _sc_bias_kernel(
    CH, n_chunks, at_hbm, wvc_hbm, out_hbm,
    b0a, b1a, b0b, b1b, acc_buf, wv_buf, sem_a, sem_b, sem_a2, sem_b2,
):
    wid = lax.axis_index("s") * 2 + lax.axis_index("c")
    n_t = (n_chunks - wid + _NW - 1) // _NW

    pltpu.sync_copy(wvc_hbm, wv_buf)
    wv_all = wv_buf[...]

    def _splat(d):
        idx = jnp.full((16,), d, jnp.int32)
        return wv_all.at[idx].get(mode="promise_in_bounds")

    wvecs = [_splat(d) for d in range(_D)]
    cvec = _splat(_D)

    pairs = ((b0a, b1a, sem_a, sem_a2), (b0b, b1b, sem_b, sem_b2))

    def in_copies(t, pair):
        b0, b1, sem, sem2 = pair
        e0 = (t * _NW + wid) * CH
        return (
            pltpu.make_async_copy(at_hbm.at[pl.ds(0, 8), pl.ds(e0, CH)], b0, sem),
            pltpu.make_async_copy(at_hbm.at[pl.ds(8, _D - 8), pl.ds(e0, CH)], b1, sem2),
        )

    @pl.when(n_t > 0)
    def _():
        for cp in in_copies(0, pairs[0]):
            cp.start()

    per_w = -(-n_chunks // _NW)
    n_outer = -(-per_w // 2)

    def outer(k, _):
        for b in range(2):
            t = k * 2 + b

            @pl.when(t + 1 < n_t)
            def _():
                for cp in in_copies(t + 1, pairs[1 - b]):
                    cp.start()

            @pl.when(t < n_t)
            def _():
                b0, b1 = pairs[b][0], pairs[b][1]
                for cp in in_copies(t, pairs[b]):
                    cp.wait()

                def group(g):
                    acc = cvec
                    for d in range(8):
                        acc = acc + b0[d, pl.ds(g, 16)] * wvecs[d]
                    for d in range(_D - 8):
                        acc = acc + b1[d, pl.ds(g, 16)] * wvecs[8 + d]
                    acc_buf[pl.ds(g, 16)] = acc

                plsc.parallel_loop(0, CH, step=16, unroll=4)(group)
                e0 = (t * _NW + wid) * CH
                pltpu.sync_copy(acc_buf, out_hbm.at[pl.ds(e0, CH)])

        return 0

    lax.fori_loop(0, n_outer, outer, 0)


def _sc_part(At, wvc, E_sc):
    n_chunks = E_sc // _CH
    mesh = plsc.VectorSubcoreMesh(
        core_axis_name="c", subcore_axis_name="s", num_cores=2, num_subcores=16
    )
    k = pl.kernel(
        functools.partial(_sc_bias_kernel, _CH, n_chunks),
        mesh=mesh,
        out_type=jax.ShapeDtypeStruct((E_sc,), jnp.float32),
        scratch_types=[
            pltpu.VMEM((8, _CH), jnp.float32),
            pltpu.VMEM((_D - 8, _CH), jnp.float32),
            pltpu.VMEM((8, _CH), jnp.float32),
            pltpu.VMEM((_D - 8, _CH), jnp.float32),
            pltpu.VMEM((_CH,), jnp.float32),
            pltpu.VMEM((16,), jnp.float32),
            pltpu.SemaphoreType.DMA,
            pltpu.SemaphoreType.DMA,
            pltpu.SemaphoreType.DMA,
            pltpu.SemaphoreType.DMA,
        ],
        compiler_params=pltpu.CompilerParams(needs_layout_passes=False),
    )
    return k(At, wvc)


def _tc_body(a_ref, w_ref, c_ref, o_ref):
    o_ref[...] = jnp.sum(a_ref[...] * w_ref[...], axis=0) + c_ref[0, 0]


def _tc_part(At, wcol, c2d, blk0, n_blk, BLKL):
    D = At.shape[0]
    return pl.pallas_call(
        _tc_body,
        grid=(n_blk,),
        in_specs=[
            pl.BlockSpec((D, BLKL), lambda i: (0, i + blk0)),
            pl.BlockSpec((D, 1), lambda i: (0, 0)),
            pl.BlockSpec(memory_space=pltpu.SMEM),
        ],
        out_specs=pl.BlockSpec((BLKL,), lambda i: (i,)),
        out_shape=jax.ShapeDtypeStruct((n_blk * BLKL,), jnp.float32),
        compiler_params=pltpu.CompilerParams(
            dimension_semantics=("arbitrary",),
        ),
    )(At, wcol, c2d)


def kernel(edge_attr, W_edge, b_edge, edge_index, n_nodes, batch):
    E, D = edge_attr.shape
    if E == 0:
        return jnp.zeros((0,), dtype=jnp.float32)

    wv = jnp.mean(W_edge, axis=1)
    c = jnp.mean(b_edge)
    wvc = jnp.concatenate([wv, c[None], jnp.zeros((16 - D - 1,), jnp.float32)])
    wcol = wv[:, None]
    c2d = c.reshape(1, 1)

    At = edge_attr.T

    BLKL = 0
    for cand in range(131072, 1023, -1024):
        if E % cand == 0 and cand % _CH == 0:
            BLKL = cand
            break
    if BLKL == 0:
        for cand in range(131072, 1023, -1024):
            if E % cand == 0:
                return _tc_part(At, wcol, c2d, 0, E // cand, cand)
        out = jnp.dot(edge_attr, wv, preferred_element_type=jnp.float32)
        return out + c

    n_blk = E // BLKL
    k_sc = (n_blk * 2) // 5
    if k_sc == 0 or k_sc == n_blk:
        return _tc_part(At, wcol, c2d, 0, n_blk, BLKL)

    out_sc = _sc_part(At, wvc, k_sc * BLKL)
    out_tc = _tc_part(At, wcol, c2d, k_sc, n_blk - k_sc, BLKL)
    return jnp.concatenate([out_sc, out_tc])
